# Initial kernel scaffold; baseline (speedup 1.0000x reference)
import jax
import jax.numpy as jnp
from jax.experimental import pallas as pl


def kernel(logits, target, alpha):
    raise NotImplementedError("write your pallas kernel here")


# speedup vs baseline: 1.0253x; 1.0253x over previous
import functools

import jax
import jax.numpy as jnp
from jax.experimental import pallas as pl
from jax.experimental.pallas import tpu as pltpu


def _focal_kernel(x_ref, t_ref, a_ref, o_ref, acc_ref, *,
                  n_classes, t_hw, hw, hw_blocks, total_blocks,
                  blocks_per_core, neg_inv_count):
    p = pl.program_id(0)
    i = pl.program_id(1)

    @pl.when(i == 0)
    def _():
        acc_ref[...] = jnp.zeros_like(acc_ref)

    x = x_ref[0].astype(jnp.float32)
    t = t_ref[0]
    a = a_ref[...]

    m = jnp.max(x, axis=0, keepdims=True)
    z = x - m
    s = jnp.sum(jnp.exp(z), axis=0, keepdims=True)

    onehot = jax.lax.broadcasted_iota(jnp.int32, (n_classes, t_hw), 0) == t
    z_t = jnp.sum(jnp.where(onehot, z, 0.0), axis=0, keepdims=True)
    a_t = jnp.sum(jnp.where(onehot, a, 0.0), axis=0, keepdims=True)

    logpt = z_t - jnp.log(s)
    pt = jnp.exp(logpt)
    one_m_pt = 1.0 - pt
    contrib = (one_m_pt * one_m_pt) * (a_t * logpt)

    ragged = (hw % t_hw) != 0
    overshoot = (2 * blocks_per_core) != total_blocks
    if not ragged and not overshoot:
        acc_ref[...] += contrib
    else:
        flat = p * blocks_per_core + i
        lane = jax.lax.broadcasted_iota(jnp.int32, (1, t_hw), 1)
        valid = jnp.logical_and(flat < total_blocks,
                                (flat % hw_blocks) * t_hw + lane < hw)
        acc_ref[...] += jnp.where(valid, contrib, 0.0)

    @pl.when(i == pl.num_programs(1) - 1)
    def _():
        o_ref[...] = (jnp.sum(acc_ref[...]) * neg_inv_count).reshape(1, 1, 1)


def _lane_tile(hw, n_classes, itemsize):
    target = 1024 * 1024
    t = (target // max(1, n_classes * itemsize)) // 128 * 128
    t = max(128, min(int(t), 65536))
    if hw % 128 == 0:
        while t > 128 and hw % t != 0:
            t -= 128
        return min(t, hw)
    if hw <= t:
        return hw
    return t


def kernel(logits, target, alpha):
    if logits.ndim > 2:
        n, c = logits.shape[0], logits.shape[1]
        hw = 1
        for d in logits.shape[2:]:
            hw *= d
        x3 = logits.reshape(n, c, hw)
        t3 = target.reshape(n, 1, hw).astype(jnp.int32)
    else:
        mrows, c = logits.shape
        n, hw = 1, mrows
        x3 = logits.T.reshape(1, c, hw)
        t3 = target.reshape(1, 1, hw).astype(jnp.int32)

    a_col = jnp.asarray(alpha, jnp.float32).reshape(-1, 1)

    itemsize = jnp.dtype(x3.dtype).itemsize
    t_hw = _lane_tile(hw, c, itemsize)
    hw_blocks = pl.cdiv(hw, t_hw)
    total_blocks = n * hw_blocks
    blocks_per_core = pl.cdiv(total_blocks, 2)

    def in_index(p, i):
        flat = jnp.minimum(p * blocks_per_core + i, total_blocks - 1)
        return (flat // hw_blocks, 0, flat % hw_blocks)

    kfn = functools.partial(
        _focal_kernel, n_classes=c, t_hw=t_hw, hw=hw, hw_blocks=hw_blocks,
        total_blocks=total_blocks, blocks_per_core=blocks_per_core,
        neg_inv_count=-1.0 / (n * hw))

    partials = pl.pallas_call(
        kfn,
        out_shape=jax.ShapeDtypeStruct((2, 1, 1), jnp.float32),
        grid_spec=pltpu.PrefetchScalarGridSpec(
            num_scalar_prefetch=0,
            grid=(2, blocks_per_core),
            in_specs=[
                pl.BlockSpec((1, c, t_hw), in_index),
                pl.BlockSpec((1, 1, t_hw), in_index),
                pl.BlockSpec((c, 1), lambda p, i: (0, 0)),
            ],
            out_specs=pl.BlockSpec((1, 1, 1), lambda p, i: (p, 0, 0)),
            scratch_shapes=[pltpu.VMEM((1, t_hw), jnp.float32)],
        ),
        compiler_params=pltpu.CompilerParams(
            dimension_semantics=("parallel", "arbitrary")),
    )(x3, t3, a_col)

    return jnp.sum(partials)


# speedup vs baseline: 3.6014x; 3.5125x over previous
import functools

import jax
import jax.numpy as jnp
from jax.experimental import pallas as pl
from jax.experimental.pallas import tpu as pltpu


def _focal_rows_kernel(a_sref, x_ref, t_ref, o_ref, acc_ref, *,
                       n_classes, rows, r_chunk, neg_inv_count):
    i = pl.program_id(1)

    @pl.when(i == 0)
    def _():
        acc_ref[...] = jnp.zeros_like(acc_ref)

    for g in range(rows // r_chunk):
        sl = pl.ds(g * r_chunk, r_chunk)
        t = t_ref[0, sl, :]
        m = x_ref[0, 0, sl, :]
        for c in range(1, n_classes):
            m = jnp.maximum(m, x_ref[0, c, sl, :])
        s = jnp.zeros((r_chunk, 128), jnp.float32)
        z_t = jnp.zeros((r_chunk, 128), jnp.float32)
        a_t = jnp.zeros((r_chunk, 128), jnp.float32)
        for c in range(n_classes):
            z_c = x_ref[0, c, sl, :] - m
            s = s + jnp.exp(z_c)
            hit = t == c
            z_t = jnp.where(hit, z_c, z_t)
            a_t = jnp.where(hit, a_sref[c], a_t)
        logpt = z_t - jnp.log(s)
        pt = jnp.exp(logpt)
        one_m_pt = 1.0 - pt
        acc_ref[sl, :] += (one_m_pt * one_m_pt) * (a_t * logpt)

    @pl.when(i == pl.num_programs(1) - 1)
    def _():
        o_ref[...] = (jnp.sum(acc_ref[...]) * neg_inv_count).reshape(1, 1, 1)


def _rows_path(x4, t3, a_vec, n, c, rows, neg_inv_count):
    r_b = rows
    row_blocks = rows // r_b
    total_blocks = n * row_blocks
    bpc = total_blocks // 2
    r_chunk = min(32, r_b)
    while r_b % r_chunk != 0:
        r_chunk //= 2

    def x_index(p, i, a_sref):
        flat = p * bpc + i
        return (flat // row_blocks, 0, flat % row_blocks, 0)

    def t_index(p, i, a_sref):
        flat = p * bpc + i
        return (flat // row_blocks, flat % row_blocks, 0)

    kfn = functools.partial(
        _focal_rows_kernel, n_classes=c, rows=r_b, r_chunk=r_chunk,
        neg_inv_count=neg_inv_count)

    partials = pl.pallas_call(
        kfn,
        out_shape=jax.ShapeDtypeStruct((2, 1, 1), jnp.float32),
        grid_spec=pltpu.PrefetchScalarGridSpec(
            num_scalar_prefetch=1,
            grid=(2, bpc),
            in_specs=[
                pl.BlockSpec((1, c, r_b, 128), x_index),
                pl.BlockSpec((1, r_b, 128), t_index),
            ],
            out_specs=pl.BlockSpec((1, 1, 1), lambda p, i, a: (p, 0, 0)),
            scratch_shapes=[pltpu.VMEM((r_b, 128), jnp.float32)],
        ),
        compiler_params=pltpu.CompilerParams(
            dimension_semantics=("parallel", "arbitrary")),
    )(a_vec, x4, t3)

    return jnp.sum(partials)


def _focal_lanes_kernel(x_ref, t_ref, a_ref, o_ref, acc_ref, *,
                        n_classes, t_hw, hw, hw_blocks, total_blocks,
                        blocks_per_core, neg_inv_count):
    p = pl.program_id(0)
    i = pl.program_id(1)

    @pl.when(i == 0)
    def _():
        acc_ref[...] = jnp.zeros_like(acc_ref)

    x = x_ref[0].astype(jnp.float32)
    t = t_ref[0]
    a = a_ref[...]

    m = jnp.max(x, axis=0, keepdims=True)
    z = x - m
    s = jnp.sum(jnp.exp(z), axis=0, keepdims=True)

    onehot = jax.lax.broadcasted_iota(jnp.int32, (n_classes, t_hw), 0) == t
    z_t = jnp.sum(jnp.where(onehot, z, 0.0), axis=0, keepdims=True)
    a_t = jnp.sum(jnp.where(onehot, a, 0.0), axis=0, keepdims=True)

    logpt = z_t - jnp.log(s)
    pt = jnp.exp(logpt)
    one_m_pt = 1.0 - pt
    contrib = (one_m_pt * one_m_pt) * (a_t * logpt)

    ragged = (hw % t_hw) != 0
    overshoot = (2 * blocks_per_core) != total_blocks
    if not ragged and not overshoot:
        acc_ref[...] += contrib
    else:
        flat = p * blocks_per_core + i
        lane = jax.lax.broadcasted_iota(jnp.int32, (1, t_hw), 1)
        valid = jnp.logical_and(flat < total_blocks,
                                (flat % hw_blocks) * t_hw + lane < hw)
        acc_ref[...] += jnp.where(valid, contrib, 0.0)

    @pl.when(i == pl.num_programs(1) - 1)
    def _():
        o_ref[...] = (jnp.sum(acc_ref[...]) * neg_inv_count).reshape(1, 1, 1)


def _lane_tile(hw, n_classes, itemsize):
    target = 1024 * 1024
    t = (target // max(1, n_classes * itemsize)) // 128 * 128
    t = max(128, min(int(t), 65536))
    if hw % 128 == 0:
        while t > 128 and hw % t != 0:
            t -= 128
        return min(t, hw)
    if hw <= t:
        return hw
    return t


def _lanes_path(x3, t3, a_col, n, c, hw, neg_inv_count):
    itemsize = jnp.dtype(x3.dtype).itemsize
    t_hw = _lane_tile(hw, c, itemsize)
    hw_blocks = pl.cdiv(hw, t_hw)
    total_blocks = n * hw_blocks
    blocks_per_core = pl.cdiv(total_blocks, 2)

    def in_index(p, i):
        flat = jnp.minimum(p * blocks_per_core + i, total_blocks - 1)
        return (flat // hw_blocks, 0, flat % hw_blocks)

    kfn = functools.partial(
        _focal_lanes_kernel, n_classes=c, t_hw=t_hw, hw=hw,
        hw_blocks=hw_blocks, total_blocks=total_blocks,
        blocks_per_core=blocks_per_core, neg_inv_count=neg_inv_count)

    partials = pl.pallas_call(
        kfn,
        out_shape=jax.ShapeDtypeStruct((2, 1, 1), jnp.float32),
        grid_spec=pltpu.PrefetchScalarGridSpec(
            num_scalar_prefetch=0,
            grid=(2, blocks_per_core),
            in_specs=[
                pl.BlockSpec((1, c, t_hw), in_index),
                pl.BlockSpec((1, 1, t_hw), in_index),
                pl.BlockSpec((c, 1), lambda p, i: (0, 0)),
            ],
            out_specs=pl.BlockSpec((1, 1, 1), lambda p, i: (p, 0, 0)),
            scratch_shapes=[pltpu.VMEM((1, t_hw), jnp.float32)],
        ),
        compiler_params=pltpu.CompilerParams(
            dimension_semantics=("parallel", "arbitrary")),
    )(x3, t3, a_col)

    return jnp.sum(partials)


def kernel(logits, target, alpha):
    if logits.ndim > 2:
        n, c = logits.shape[0], logits.shape[1]
        hw = 1
        for d in logits.shape[2:]:
            hw *= d
        x3 = logits.reshape(n, c, hw)
        t3 = target.reshape(n, 1, hw).astype(jnp.int32)
    else:
        mrows, c = logits.shape
        n, hw = 1, mrows
        x3 = logits.T.reshape(1, c, hw)
        t3 = target.reshape(1, 1, hw).astype(jnp.int32)

    a_vec = jnp.asarray(alpha, jnp.float32).reshape(-1)
    neg_inv_count = -1.0 / (n * hw)

    use_rows = (hw % 128 == 0 and n % 2 == 0 and
                x3.dtype == jnp.float32 and 2 <= c <= 128 and
                c * hw * 4 <= 4 * 1024 * 1024)
    if use_rows:
        rows = hw // 128
        x4 = x3.reshape(n, c, rows, 128)
        t4 = t3.reshape(n, rows, 128)
        return _rows_path(x4, t4, a_vec, n, c, rows, neg_inv_count)
    return _lanes_path(x3, t3, a_vec.reshape(-1, 1), n, c, hw, neg_inv_count)


# speedup vs baseline: 4.9967x; 1.3874x over previous
import functools

import jax
import jax.numpy as jnp
from jax.experimental import pallas as pl
from jax.experimental.pallas import tpu as pltpu


def _focal_rows_kernel(a_sref, x_ref, t_ref, o_ref, acc_ref, *,
                       n_classes, rows, r_chunk, img_per_blk, neg_inv_count):
    i = pl.program_id(1)

    @pl.when(i == 0)
    def _():
        acc_ref[...] = jnp.zeros_like(acc_ref)

    for b in range(img_per_blk):
        for g in range(rows // r_chunk):
            sl = pl.ds(g * r_chunk, r_chunk)
            t = t_ref[b, sl, :]
            m = x_ref[b, 0, sl, :]
            for c in range(1, n_classes):
                m = jnp.maximum(m, x_ref[b, c, sl, :])
            s = jnp.zeros((r_chunk, 128), jnp.float32)
            z_t = jnp.zeros((r_chunk, 128), jnp.float32)
            a_t = jnp.zeros((r_chunk, 128), jnp.float32)
            for c in range(n_classes):
                z_c = x_ref[b, c, sl, :] - m
                s = s + jnp.exp(z_c)
                hit = t == c
                z_t = jnp.where(hit, z_c, z_t)
                a_t = jnp.where(hit, a_sref[c], a_t)
            logpt = z_t - jnp.log(s)
            pt = jnp.exp(logpt)
            one_m_pt = 1.0 - pt
            acc_ref[sl, :] += (one_m_pt * one_m_pt) * (a_t * logpt)

    @pl.when(i == pl.num_programs(1) - 1)
    def _():
        o_ref[...] = (jnp.sum(acc_ref[...]) * neg_inv_count).reshape(1, 1, 1)


def _rows_path(x4, t3, a_vec, n, c, rows, neg_inv_count):
    r_b = rows
    row_blocks = rows // r_b
    ipb = 1
    for cand in (4, 2):
        if row_blocks == 1 and n % (2 * cand) == 0:
            ipb = cand
            break
    total_blocks = (n // ipb) * row_blocks
    bpc = total_blocks // 2
    r_chunk = min(32, r_b)
    while r_b % r_chunk != 0:
        r_chunk //= 2

    def x_index(p, i, a_sref):
        flat = p * bpc + i
        return (flat // row_blocks, 0, flat % row_blocks, 0)

    def t_index(p, i, a_sref):
        flat = p * bpc + i
        return (flat // row_blocks, flat % row_blocks, 0)

    kfn = functools.partial(
        _focal_rows_kernel, n_classes=c, rows=r_b, r_chunk=r_chunk,
        img_per_blk=ipb, neg_inv_count=neg_inv_count)

    partials = pl.pallas_call(
        kfn,
        out_shape=jax.ShapeDtypeStruct((2, 1, 1), jnp.float32),
        grid_spec=pltpu.PrefetchScalarGridSpec(
            num_scalar_prefetch=1,
            grid=(2, bpc),
            in_specs=[
                pl.BlockSpec((ipb, c, r_b, 128), x_index),
                pl.BlockSpec((ipb, r_b, 128), t_index),
            ],
            out_specs=pl.BlockSpec((1, 1, 1), lambda p, i, a: (p, 0, 0)),
            scratch_shapes=[pltpu.VMEM((r_b, 128), jnp.float32)],
        ),
        compiler_params=pltpu.CompilerParams(
            dimension_semantics=("parallel", "arbitrary")),
    )(a_vec, x4, t3)

    return jnp.sum(partials)


def _focal_lanes_kernel(x_ref, t_ref, a_ref, o_ref, acc_ref, *,
                        n_classes, t_hw, hw, hw_blocks, total_blocks,
                        blocks_per_core, neg_inv_count):
    p = pl.program_id(0)
    i = pl.program_id(1)

    @pl.when(i == 0)
    def _():
        acc_ref[...] = jnp.zeros_like(acc_ref)

    x = x_ref[0].astype(jnp.float32)
    t = t_ref[0]
    a = a_ref[...]

    m = jnp.max(x, axis=0, keepdims=True)
    z = x - m
    s = jnp.sum(jnp.exp(z), axis=0, keepdims=True)

    onehot = jax.lax.broadcasted_iota(jnp.int32, (n_classes, t_hw), 0) == t
    z_t = jnp.sum(jnp.where(onehot, z, 0.0), axis=0, keepdims=True)
    a_t = jnp.sum(jnp.where(onehot, a, 0.0), axis=0, keepdims=True)

    logpt = z_t - jnp.log(s)
    pt = jnp.exp(logpt)
    one_m_pt = 1.0 - pt
    contrib = (one_m_pt * one_m_pt) * (a_t * logpt)

    ragged = (hw % t_hw) != 0
    overshoot = (2 * blocks_per_core) != total_blocks
    if not ragged and not overshoot:
        acc_ref[...] += contrib
    else:
        flat = p * blocks_per_core + i
        lane = jax.lax.broadcasted_iota(jnp.int32, (1, t_hw), 1)
        valid = jnp.logical_and(flat < total_blocks,
                                (flat % hw_blocks) * t_hw + lane < hw)
        acc_ref[...] += jnp.where(valid, contrib, 0.0)

    @pl.when(i == pl.num_programs(1) - 1)
    def _():
        o_ref[...] = (jnp.sum(acc_ref[...]) * neg_inv_count).reshape(1, 1, 1)


def _lane_tile(hw, n_classes, itemsize):
    target = 1024 * 1024
    t = (target // max(1, n_classes * itemsize)) // 128 * 128
    t = max(128, min(int(t), 65536))
    if hw % 128 == 0:
        while t > 128 and hw % t != 0:
            t -= 128
        return min(t, hw)
    if hw <= t:
        return hw
    return t


def _lanes_path(x3, t3, a_col, n, c, hw, neg_inv_count):
    itemsize = jnp.dtype(x3.dtype).itemsize
    t_hw = _lane_tile(hw, c, itemsize)
    hw_blocks = pl.cdiv(hw, t_hw)
    total_blocks = n * hw_blocks
    blocks_per_core = pl.cdiv(total_blocks, 2)

    def in_index(p, i):
        flat = jnp.minimum(p * blocks_per_core + i, total_blocks - 1)
        return (flat // hw_blocks, 0, flat % hw_blocks)

    kfn = functools.partial(
        _focal_lanes_kernel, n_classes=c, t_hw=t_hw, hw=hw,
        hw_blocks=hw_blocks, total_blocks=total_blocks,
        blocks_per_core=blocks_per_core, neg_inv_count=neg_inv_count)

    partials = pl.pallas_call(
        kfn,
        out_shape=jax.ShapeDtypeStruct((2, 1, 1), jnp.float32),
        grid_spec=pltpu.PrefetchScalarGridSpec(
            num_scalar_prefetch=0,
            grid=(2, blocks_per_core),
            in_specs=[
                pl.BlockSpec((1, c, t_hw), in_index),
                pl.BlockSpec((1, 1, t_hw), in_index),
                pl.BlockSpec((c, 1), lambda p, i: (0, 0)),
            ],
            out_specs=pl.BlockSpec((1, 1, 1), lambda p, i: (p, 0, 0)),
            scratch_shapes=[pltpu.VMEM((1, t_hw), jnp.float32)],
        ),
        compiler_params=pltpu.CompilerParams(
            dimension_semantics=("parallel", "arbitrary")),
    )(x3, t3, a_col)

    return jnp.sum(partials)


def kernel(logits, target, alpha):
    if logits.ndim > 2:
        n, c = logits.shape[0], logits.shape[1]
        hw = 1
        for d in logits.shape[2:]:
            hw *= d
        x3 = logits.reshape(n, c, hw)
        t3 = target.reshape(n, 1, hw).astype(jnp.int32)
    else:
        mrows, c = logits.shape
        n, hw = 1, mrows
        x3 = logits.T.reshape(1, c, hw)
        t3 = target.reshape(1, 1, hw).astype(jnp.int32)

    a_vec = jnp.asarray(alpha, jnp.float32).reshape(-1)
    neg_inv_count = -1.0 / (n * hw)

    use_rows = (hw % 128 == 0 and n % 2 == 0 and
                x3.dtype == jnp.float32 and 2 <= c <= 128 and
                c * hw * 4 <= 4 * 1024 * 1024)
    if use_rows:
        rows = hw // 128
        x4 = x3.reshape(n, c, rows, 128)
        t4 = t3.reshape(n, rows, 128)
        return _rows_path(x4, t4, a_vec, n, c, rows, neg_inv_count)
    return _lanes_path(x3, t3, a_vec.reshape(-1, 1), n, c, hw, neg_inv_count)
